# Initial kernel scaffold; baseline (speedup 1.0000x reference)
import functools

import jax
import jax.numpy as jnp
from jax.experimental import pallas as pl

_E = 8
_KH = _KW = 3


def _pool_kernel(x_ref, out_ref):
    i = pl.program_id(1)

    @pl.when(i == 0)
    def _():
        out_ref[...] = jnp.zeros_like(out_ref)

    out_ref[...] += jnp.sum(x_ref[...], axis=(1, 2))


def _gate_kernel(pooled_ref, gwt_ref, gb_ref, ew_ref, eb_ref, k_ref,
                 cw_ref, cb_ref, *, n_pixels):
    pooled = pooled_ref[...] * (1.0 / n_pixels)
    b = pooled.shape[0]
    logits = jnp.sum(pooled[:, None, :] * gwt_ref[...][None, :, :], axis=2)
    logits = logits + gb_ref[...]
    m = jnp.max(logits, axis=1, keepdims=True)
    ex = jnp.exp(logits - m)
    w = ex / jnp.sum(ex, axis=1, keepdims=True)

    iota = jax.lax.broadcasted_iota(jnp.int32, w.shape, 1)
    m1 = jnp.max(w, axis=1, keepdims=True)
    i1 = jnp.min(jnp.where(w == m1, iota, _E), axis=1, keepdims=True)
    w2 = jnp.where(iota == i1, -jnp.inf, w)
    m2 = jnp.max(w2, axis=1, keepdims=True)
    i2 = jnp.min(jnp.where(w2 == m2, iota, _E), axis=1, keepdims=True)
    ew = jnp.where(iota == i1, m1, 0.0) + jnp.where(iota == i2, m2, 0.0)
    scale = ew * k_ref[0, 0]

    for bi in range(b):
        acc_w = scale[bi, 0] * ew_ref[0]
        acc_b = scale[bi, 0] * eb_ref[0:1, :]
        for e in range(1, _E):
            acc_w = acc_w + scale[bi, e] * ew_ref[e]
            acc_b = acc_b + scale[bi, e] * eb_ref[e:e + 1, :]
        cw_ref[bi] = acc_w
        cb_ref[bi:bi + 1, :] = acc_b


def _conv_kernel(xp_ref, xr_ref, w_ref, b_ref, out_ref, *, th, width, chan):
    i = pl.program_id(1)
    row0 = i * th
    acc = jnp.zeros((th * width, chan), jnp.float32)
    for dy in range(_KH):
        xs_dy = xp_ref[0, pl.ds(row0 + dy, th), :, :]
        for dx in range(_KW):
            xs = xs_dy[:, dx:dx + width, :].reshape(th * width, chan)
            wtap = w_ref[0, _KW * dy + dx, :, :].astype(jnp.bfloat16)
            acc = acc + jax.lax.dot_general(
                xs, wtap, (((1,), (0,)), ((), ())),
                preferred_element_type=jnp.float32)
    resid = xr_ref[...].reshape(th * width, chan)
    out = acc + resid + b_ref[...]
    out_ref[...] = out.reshape(1, th, width, chan)


def kernel(inputs, k, expert_w, expert_b, gate_w, gate_b):
    bsz, chan, height, width = inputs.shape
    n_pixels = height * width

    x_t = inputs.transpose(0, 2, 3, 1)
    x_pad = jnp.pad(x_t, ((0, 0), (1, 1), (1, 1), (0, 0))).astype(jnp.bfloat16)

    ew9 = expert_w.transpose(0, 3, 4, 2, 1).reshape(_E, _KH * _KW, chan, chan)
    gwt = gate_w.T
    gb2 = gate_b.reshape(1, _E)
    k2 = k.reshape(1, 1)

    th_p = 56
    ni_p = height // th_p
    pooled = pl.pallas_call(
        _pool_kernel,
        grid=(bsz, ni_p),
        in_specs=[pl.BlockSpec((1, th_p, width, chan), lambda b, i: (b, i, 0, 0))],
        out_specs=pl.BlockSpec((1, chan), lambda b, i: (b, 0)),
        out_shape=jax.ShapeDtypeStruct((bsz, chan), jnp.float32),
    )(x_t)

    cw, cb = pl.pallas_call(
        functools.partial(_gate_kernel, n_pixels=n_pixels),
        in_specs=[
            pl.BlockSpec(pooled.shape, lambda: (0, 0)),
            pl.BlockSpec(gwt.shape, lambda: (0, 0)),
            pl.BlockSpec(gb2.shape, lambda: (0, 0)),
            pl.BlockSpec(ew9.shape, lambda: (0, 0, 0, 0)),
            pl.BlockSpec(expert_b.shape, lambda: (0, 0)),
            pl.BlockSpec(k2.shape, lambda: (0, 0)),
        ],
        out_specs=[
            pl.BlockSpec((bsz, _KH * _KW, chan, chan), lambda: (0, 0, 0, 0)),
            pl.BlockSpec((bsz, chan), lambda: (0, 0)),
        ],
        out_shape=[
            jax.ShapeDtypeStruct((bsz, _KH * _KW, chan, chan), jnp.float32),
            jax.ShapeDtypeStruct((bsz, chan), jnp.float32),
        ],
    )(pooled, gwt, gb2, ew9, expert_b, k2)

    th = 28
    ni = height // th
    out_t = pl.pallas_call(
        functools.partial(_conv_kernel, th=th, width=width, chan=chan),
        grid=(bsz, ni),
        in_specs=[
            pl.BlockSpec((1, height + 2, width + 2, chan), lambda b, i: (b, 0, 0, 0)),
            pl.BlockSpec((1, th, width, chan), lambda b, i: (b, i, 0, 0)),
            pl.BlockSpec((1, _KH * _KW, chan, chan), lambda b, i: (b, 0, 0, 0)),
            pl.BlockSpec((1, chan), lambda b, i: (b, 0)),
        ],
        out_specs=pl.BlockSpec((1, th, width, chan), lambda b, i: (b, i, 0, 0)),
        out_shape=jax.ShapeDtypeStruct((bsz, height, width, chan), jnp.float32),
    )(x_pad, x_t, cw, cb)

    return out_t.transpose(0, 3, 1, 2)


# speedup vs baseline: 4.3406x; 4.3406x over previous
import functools

import jax
import jax.numpy as jnp
from jax.experimental import pallas as pl

_E = 8
_KH = _KW = 3


def _pool_kernel(x_ref, out_ref):
    i = pl.program_id(1)

    @pl.when(i == 0)
    def _():
        out_ref[...] = jnp.zeros_like(out_ref)

    out_ref[...] += jnp.sum(x_ref[...], axis=(1, 2))[:, None, :]


def _gate_kernel(pooled_ref, gwt_ref, gb_ref, ew_ref, eb_ref, k_ref,
                 cw_ref, cb_ref, *, n_pixels):
    pooled = pooled_ref[:, 0, :] * (1.0 / n_pixels)
    b = pooled.shape[0]
    logits = jnp.sum(pooled[:, None, :] * gwt_ref[...][None, :, :], axis=2)
    logits = logits + gb_ref[...]
    m = jnp.max(logits, axis=1, keepdims=True)
    ex = jnp.exp(logits - m)
    w = ex / jnp.sum(ex, axis=1, keepdims=True)

    iota = jax.lax.broadcasted_iota(jnp.int32, w.shape, 1)
    m1 = jnp.max(w, axis=1, keepdims=True)
    i1 = jnp.min(jnp.where(w == m1, iota, _E), axis=1, keepdims=True)
    w2 = jnp.where(iota == i1, -jnp.inf, w)
    m2 = jnp.max(w2, axis=1, keepdims=True)
    i2 = jnp.min(jnp.where(w2 == m2, iota, _E), axis=1, keepdims=True)
    ew = jnp.where(iota == i1, m1, 0.0) + jnp.where(iota == i2, m2, 0.0)
    scale = ew * k_ref[0, 0]

    for bi in range(b):
        acc_w = scale[bi, 0] * ew_ref[0]
        acc_b = scale[bi, 0] * eb_ref[0:1, :]
        for e in range(1, _E):
            acc_w = acc_w + scale[bi, e] * ew_ref[e]
            acc_b = acc_b + scale[bi, e] * eb_ref[e:e + 1, :]
        cw_ref[bi] = acc_w
        cb_ref[bi] = acc_b


def _conv_kernel(xp_ref, xr_ref, w_ref, b_ref, out_ref, *, th, width, chan):
    i = pl.program_id(1)
    row0 = i * th
    acc = jnp.zeros((th * width, chan), jnp.float32)
    for dy in range(_KH):
        xs_dy = xp_ref[0, pl.ds(row0 + dy, th), :, :]
        for dx in range(_KW):
            xs = xs_dy[:, dx:dx + width, :].reshape(th * width, chan)
            wtap = w_ref[0, _KW * dy + dx, :, :].astype(jnp.bfloat16)
            acc = acc + jax.lax.dot_general(
                xs, wtap, (((1,), (0,)), ((), ())),
                preferred_element_type=jnp.float32)
    resid = xr_ref[...].reshape(th * width, chan)
    out = acc + resid + b_ref[0]
    out_ref[...] = out.reshape(1, th, width, chan)


def kernel(inputs, k, expert_w, expert_b, gate_w, gate_b):
    bsz, chan, height, width = inputs.shape
    n_pixels = height * width

    x_t = inputs.transpose(0, 2, 3, 1)
    x_pad = jnp.pad(x_t, ((0, 0), (1, 1), (1, 1), (0, 0))).astype(jnp.bfloat16)

    ew9 = expert_w.transpose(0, 3, 4, 2, 1).reshape(_E, _KH * _KW, chan, chan)
    gwt = gate_w.T
    gb2 = gate_b.reshape(1, _E)
    k2 = k.reshape(1, 1)

    th_p = 56
    ni_p = height // th_p
    pooled = pl.pallas_call(
        _pool_kernel,
        grid=(bsz, ni_p),
        in_specs=[pl.BlockSpec((1, th_p, width, chan), lambda b, i: (b, i, 0, 0))],
        out_specs=pl.BlockSpec((1, 1, chan), lambda b, i: (b, 0, 0)),
        out_shape=jax.ShapeDtypeStruct((bsz, 1, chan), jnp.float32),
    )(x_t)

    cw, cb = pl.pallas_call(
        functools.partial(_gate_kernel, n_pixels=n_pixels),
        in_specs=[
            pl.BlockSpec(pooled.shape, lambda: (0, 0, 0)),
            pl.BlockSpec(gwt.shape, lambda: (0, 0)),
            pl.BlockSpec(gb2.shape, lambda: (0, 0)),
            pl.BlockSpec(ew9.shape, lambda: (0, 0, 0, 0)),
            pl.BlockSpec(expert_b.shape, lambda: (0, 0)),
            pl.BlockSpec(k2.shape, lambda: (0, 0)),
        ],
        out_specs=[
            pl.BlockSpec((bsz, _KH * _KW, chan, chan), lambda: (0, 0, 0, 0)),
            pl.BlockSpec((bsz, 1, chan), lambda: (0, 0, 0)),
        ],
        out_shape=[
            jax.ShapeDtypeStruct((bsz, _KH * _KW, chan, chan), jnp.float32),
            jax.ShapeDtypeStruct((bsz, 1, chan), jnp.float32),
        ],
    )(pooled, gwt, gb2, ew9, expert_b, k2)

    th = 28
    ni = height // th
    out_t = pl.pallas_call(
        functools.partial(_conv_kernel, th=th, width=width, chan=chan),
        grid=(bsz, ni),
        in_specs=[
            pl.BlockSpec((1, height + 2, width + 2, chan), lambda b, i: (b, 0, 0, 0)),
            pl.BlockSpec((1, th, width, chan), lambda b, i: (b, i, 0, 0)),
            pl.BlockSpec((1, _KH * _KW, chan, chan), lambda b, i: (b, 0, 0, 0)),
            pl.BlockSpec((1, 1, chan), lambda b, i: (b, 0, 0)),
        ],
        out_specs=pl.BlockSpec((1, th, width, chan), lambda b, i: (b, i, 0, 0)),
        out_shape=jax.ShapeDtypeStruct((bsz, height, width, chan), jnp.float32),
    )(x_pad, x_t, cw, cb)

    return out_t.transpose(0, 3, 1, 2)


# speedup vs baseline: 9.0214x; 2.0784x over previous
import functools

import jax
import jax.numpy as jnp
from jax.experimental import pallas as pl
from jax.experimental.pallas import tpu as pltpu

_E = 8
_KH = _KW = 3


def _fused_kernel(x_ref, gwt_ref, gb_ref, ew_ref, eb_ref, k_ref, out_ref,
                  x1_scr, pool_scr, w_scr, b_scr,
                  *, rb, n_rb, th, ni, width, chan, n_pixels):
    s = pl.program_id(1)
    n_fmt = n_rb + 2

    @pl.when(s == 0)
    def _():
        pool_scr[...] = jnp.zeros_like(pool_scr)

    @pl.when((s >= 1) & (s <= n_rb))
    def _():
        pool_scr[...] += jnp.sum(x_ref[...], axis=(0, 2, 3))[None, :]
        t = jnp.transpose(x_ref[0].astype(jnp.bfloat16), (1, 2, 0))
        x1_scr[pl.ds(s * rb, rb), 0:width, :] = t
        x1_scr[pl.ds(s * rb, rb), width:, :] = jnp.zeros(
            (rb, 2, chan), jnp.bfloat16)

    @pl.when((s == 0) | (s == n_rb + 1))
    def _():
        x1_scr[pl.ds(s * rb, rb), :, :] = jnp.zeros(
            (rb, width + 2, chan), jnp.bfloat16)

    @pl.when(s == n_fmt)
    def _():
        pooled = pool_scr[...] * (1.0 / n_pixels)
        logits = jnp.sum(pooled[:, None, :] * gwt_ref[...][None, :, :], axis=2)
        logits = logits + gb_ref[...]
        m = jnp.max(logits, axis=1, keepdims=True)
        ex = jnp.exp(logits - m)
        w = ex / jnp.sum(ex, axis=1, keepdims=True)

        iota = jax.lax.broadcasted_iota(jnp.int32, w.shape, 1)
        m1 = jnp.max(w, axis=1, keepdims=True)
        i1 = jnp.min(jnp.where(w == m1, iota, _E), axis=1, keepdims=True)
        w2 = jnp.where(iota == i1, -jnp.inf, w)
        m2 = jnp.max(w2, axis=1, keepdims=True)
        i2 = jnp.min(jnp.where(w2 == m2, iota, _E), axis=1, keepdims=True)
        ew = jnp.where(iota == i1, m1, 0.0) + jnp.where(iota == i2, m2, 0.0)
        scale = ew * k_ref[0, 0]

        center = _KW * (_KH // 2) + _KW // 2
        t3 = jax.lax.broadcasted_iota(jnp.int32, (_KH * _KW, chan, chan), 0)
        rr = jax.lax.broadcasted_iota(jnp.int32, (_KH * _KW, chan, chan), 1)
        cc = jax.lax.broadcasted_iota(jnp.int32, (_KH * _KW, chan, chan), 2)
        eye3 = ((t3 == center) & (rr == cc)).astype(jnp.float32)

        acc_w = scale[0, 0] * ew_ref[0]
        acc_b = scale[0, 0] * eb_ref[0:1, :]
        for e in range(1, _E):
            acc_w = acc_w + scale[0, e] * ew_ref[e]
            acc_b = acc_b + scale[0, e] * eb_ref[e:e + 1, :]
        w_scr[...] = acc_w + eye3
        b_scr[...] = acc_b

    @pl.when(s >= n_fmt + 1)
    def _():
        i = s - (n_fmt + 1)
        row0 = i * th
        acc = jnp.zeros((th * width, chan), jnp.float32)
        for dy in range(_KH):
            slab = x1_scr[pl.ds(row0 + rb - 1 + dy, th), :, :]
            left = jnp.concatenate(
                [slab[:, width + 1 :, :], slab[:, : width - 1, :]], axis=1)
            for dx, sub in ((0, left),
                            (1, slab[:, 0:width, :]),
                            (2, slab[:, 1 : width + 1, :])):
                wtap = w_scr[_KW * dy + dx, :, :].astype(jnp.bfloat16)
                acc = acc + jax.lax.dot_general(
                    sub.reshape(th * width, chan), wtap,
                    (((1,), (0,)), ((), ())),
                    preferred_element_type=jnp.float32)
        out = acc + b_scr[...]
        out_ref[...] = jnp.transpose(
            out.reshape(th, width, chan), (2, 0, 1))[None]


def kernel(inputs, k, expert_w, expert_b, gate_w, gate_b):
    bsz, chan, height, width = inputs.shape
    n_pixels = height * width
    rb = 32
    n_rb = height // rb
    th = 32
    ni = height // th
    n_fmt = n_rb + 2
    n_steps = n_fmt + 1 + ni
    hp = height + 2 * rb
    wp = width + 2

    ew9 = expert_w.transpose(0, 3, 4, 2, 1).reshape(_E, _KH * _KW, chan, chan)
    gwt = gate_w.T
    gb2 = gate_b.reshape(1, _E)
    k2 = k.reshape(1, 1)

    out = pl.pallas_call(
        functools.partial(
            _fused_kernel, rb=rb, n_rb=n_rb, th=th, ni=ni, width=width,
            chan=chan, n_pixels=n_pixels),
        grid=(bsz, n_steps),
        in_specs=[
            pl.BlockSpec((1, chan, rb, width),
                         lambda b, s: (b, 0, jnp.clip(s - 1, 0, n_rb - 1), 0)),
            pl.BlockSpec(gwt.shape, lambda b, s: (0, 0)),
            pl.BlockSpec(gb2.shape, lambda b, s: (0, 0)),
            pl.BlockSpec(ew9.shape, lambda b, s: (0, 0, 0, 0)),
            pl.BlockSpec(expert_b.shape, lambda b, s: (0, 0)),
            pl.BlockSpec(k2.shape, lambda b, s: (0, 0)),
        ],
        out_specs=pl.BlockSpec(
            (1, chan, th, width),
            lambda b, s: (b, 0, jnp.clip(s - (n_rb + 3), 0, ni - 1), 0)),
        out_shape=jax.ShapeDtypeStruct((bsz, chan, height, width), jnp.float32),
        scratch_shapes=[
            pltpu.VMEM((hp, wp, chan), jnp.bfloat16),
            pltpu.VMEM((1, chan), jnp.float32),
            pltpu.VMEM((_KH * _KW, chan, chan), jnp.float32),
            pltpu.VMEM((1, chan), jnp.float32),
        ],
    )(inputs, gwt, gb2, ew9, expert_b, k2)

    return out


# speedup vs baseline: 11.7833x; 1.3061x over previous
import functools

import jax
import jax.numpy as jnp
from jax.experimental import pallas as pl
from jax.experimental.pallas import tpu as pltpu

_E = 8
_KH = _KW = 3


def _fused_kernel(x_ref, gwt_ref, gb_ref, ew_ref, eb_ref, k_ref, out_ref,
                  x1_scr, pool_scr, w_scr, b_scr,
                  *, rb, n_rb, th, ni, width, chan, n_pixels):
    s = pl.program_id(1)
    n_fmt = n_rb + 2

    @pl.when(s == 0)
    def _():
        pool_scr[...] = jnp.zeros_like(pool_scr)

    @pl.when((s >= 1) & (s <= n_rb))
    def _():
        pool_scr[...] += jnp.sum(x_ref[...], axis=(0, 2, 3))[None, :]
        t = jnp.transpose(x_ref[0].astype(jnp.bfloat16), (1, 2, 0))
        x1_scr[pl.ds(s * rb, rb), 0:width, :] = t
        x1_scr[pl.ds(s * rb, rb), width:, :] = jnp.zeros(
            (rb, 8, chan), jnp.bfloat16)

    @pl.when((s == 0) | (s == n_rb + 1))
    def _():
        x1_scr[pl.ds(s * rb, rb), :, :] = jnp.zeros(
            (rb, width + 8, chan), jnp.bfloat16)

    @pl.when(s == n_fmt)
    def _():
        pooled = pool_scr[...] * (1.0 / n_pixels)
        logits = jnp.sum(pooled[:, None, :] * gwt_ref[...][None, :, :], axis=2)
        logits = logits + gb_ref[...]
        m = jnp.max(logits, axis=1, keepdims=True)
        ex = jnp.exp(logits - m)
        w = ex / jnp.sum(ex, axis=1, keepdims=True)

        iota = jax.lax.broadcasted_iota(jnp.int32, w.shape, 1)
        m1 = jnp.max(w, axis=1, keepdims=True)
        i1 = jnp.min(jnp.where(w == m1, iota, _E), axis=1, keepdims=True)
        w2 = jnp.where(iota == i1, -jnp.inf, w)
        m2 = jnp.max(w2, axis=1, keepdims=True)
        i2 = jnp.min(jnp.where(w2 == m2, iota, _E), axis=1, keepdims=True)
        ew = jnp.where(iota == i1, m1, 0.0) + jnp.where(iota == i2, m2, 0.0)
        scale = ew * k_ref[0, 0]

        center = _KW * (_KH // 2) + _KW // 2
        t3 = jax.lax.broadcasted_iota(jnp.int32, (_KH * _KW, chan, chan), 0)
        rr = jax.lax.broadcasted_iota(jnp.int32, (_KH * _KW, chan, chan), 1)
        cc = jax.lax.broadcasted_iota(jnp.int32, (_KH * _KW, chan, chan), 2)
        eye3 = ((t3 == center) & (rr == cc)).astype(jnp.float32)

        acc_w = scale[0, 0] * ew_ref[0]
        acc_b = scale[0, 0] * eb_ref[0:1, :]
        for e in range(1, _E):
            acc_w = acc_w + scale[0, e] * ew_ref[e]
            acc_b = acc_b + scale[0, e] * eb_ref[e:e + 1, :]
        w_scr[...] = acc_w + eye3
        b_scr[...] = acc_b

    @pl.when(s >= n_fmt + 1)
    def _():
        i = s - (n_fmt + 1)
        row0 = i * th
        wf = width + 8
        m = th * wf
        dn = (((1,), (0,)), ((), ()))
        accs = [jnp.zeros((m, chan), jnp.float32) for _ in range(_KW)]
        for dy in range(_KH):
            slab = x1_scr[pl.ds(row0 + rb - 1 + dy, th), :, :]
            flat = slab.reshape(m, chan)
            for dx in range(_KW):
                wtap = w_scr[_KW * dy + dx, :, :].astype(jnp.bfloat16)
                accs[dx] = accs[dx] + jax.lax.dot_general(
                    flat, wtap, dn, preferred_element_type=jnp.float32)
        zrow = jnp.zeros((1, chan), jnp.float32)
        out = (accs[1]
               + jnp.concatenate([zrow, accs[0][:-1, :]], axis=0)
               + jnp.concatenate([accs[2][1:, :], zrow], axis=0))
        out = out.reshape(th, wf, chan)[:, 0:width, :] + b_scr[...]
        out_ref[...] = jnp.transpose(out, (2, 0, 1))[None]


def kernel(inputs, k, expert_w, expert_b, gate_w, gate_b):
    bsz, chan, height, width = inputs.shape
    n_pixels = height * width
    rb = 32
    n_rb = height // rb
    th = 32
    ni = height // th
    n_fmt = n_rb + 2
    n_steps = n_fmt + 1 + ni
    hp = height + 2 * rb
    wp = width + 8

    ew9 = expert_w.transpose(0, 3, 4, 2, 1).reshape(_E, _KH * _KW, chan, chan)
    gwt = gate_w.T
    gb2 = gate_b.reshape(1, _E)
    k2 = k.reshape(1, 1)

    out = pl.pallas_call(
        functools.partial(
            _fused_kernel, rb=rb, n_rb=n_rb, th=th, ni=ni, width=width,
            chan=chan, n_pixels=n_pixels),
        grid=(bsz, n_steps),
        in_specs=[
            pl.BlockSpec((1, chan, rb, width),
                         lambda b, s: (b, 0, jnp.clip(s - 1, 0, n_rb - 1), 0)),
            pl.BlockSpec(gwt.shape, lambda b, s: (0, 0)),
            pl.BlockSpec(gb2.shape, lambda b, s: (0, 0)),
            pl.BlockSpec(ew9.shape, lambda b, s: (0, 0, 0, 0)),
            pl.BlockSpec(expert_b.shape, lambda b, s: (0, 0)),
            pl.BlockSpec(k2.shape, lambda b, s: (0, 0)),
        ],
        out_specs=pl.BlockSpec(
            (1, chan, th, width),
            lambda b, s: (b, 0, jnp.clip(s - (n_rb + 3), 0, ni - 1), 0)),
        out_shape=jax.ShapeDtypeStruct((bsz, chan, height, width), jnp.float32),
        scratch_shapes=[
            pltpu.VMEM((hp, wp, chan), jnp.bfloat16),
            pltpu.VMEM((1, chan), jnp.float32),
            pltpu.VMEM((_KH * _KW, chan, chan), jnp.float32),
            pltpu.VMEM((1, chan), jnp.float32),
        ],
    )(inputs, gwt, gb2, ew9, expert_b, k2)

    return out
